# Initial kernel scaffold; baseline (speedup 1.0000x reference)
import jax
import jax.numpy as jnp
from jax.experimental import pallas as pl


def kernel(u_pred, u_obs, h, lam):
    raise NotImplementedError("write your pallas kernel here")


# speedup vs baseline: 1.0297x; 1.0297x over previous
import functools
import numpy as np
import jax
import jax.numpy as jnp
from jax.experimental import pallas as pl
from jax.experimental.pallas import tpu as pltpu

_ELT = 64
_ROW = 65536
_K = 384
_ROWS_PER_SEG = _ROW * 3 // _K
_T2 = np.float32((2.0 * np.sin(np.radians(10.0) / 2.0)) ** 2)


def _make_sel():
    sel = np.zeros((_K, 128), dtype=np.float32)
    j = np.arange(128)
    for c in range(3):
        sel[3 * j + c, j] = 1.0
    return jnp.asarray(sel)


def _tc_body(sel_ref, p_ref, o_ref, h_ref, lam_ref, out_ref):
    d = p_ref[...] - o_ref[...]
    d2 = d * d
    s2 = jax.lax.dot_general(
        d2, sel_ref[...], (((1,), (0,)), ((), ())),
        precision=jax.lax.Precision.HIGHEST,
        preferred_element_type=jnp.float32,
    )
    h = h_ref[0]
    lam = lam_ref[0]
    v = s2 * (1.0 / _T2)
    p = h * jnp.exp(-lam * v) + (1.0 - h)
    log_p = jnp.where(s2 < _T2, jnp.log(p), 0.0)
    out_ref[...] = jnp.sum(log_p, dtype=jnp.float32)[None, None, None] * jnp.ones(
        (1, 1, 128), jnp.float32)


@jax.jit
def kernel(u_pred, u_obs, h, lam):
    pf = u_pred.reshape(_ELT * _ROWS_PER_SEG, _K)
    of = u_obs.reshape(_ELT * _ROWS_PER_SEG, _K)
    hb = jnp.broadcast_to(h[:, None, None], (_ELT, 1, 128))
    lb = jnp.broadcast_to(lam[:, None, None], (_ELT, 1, 128))
    sel = _make_sel()
    out = pl.pallas_call(
        _tc_body,
        grid=(_ELT,),
        in_specs=[
            pl.BlockSpec((_K, 128), lambda e: (0, 0)),
            pl.BlockSpec((_ROWS_PER_SEG, _K), lambda e: (e, 0)),
            pl.BlockSpec((_ROWS_PER_SEG, _K), lambda e: (e, 0)),
            pl.BlockSpec((1, 1, 128), lambda e: (e, 0, 0)),
            pl.BlockSpec((1, 1, 128), lambda e: (e, 0, 0)),
        ],
        out_specs=pl.BlockSpec((1, 1, 128), lambda e: (e, 0, 0)),
        out_shape=jax.ShapeDtypeStruct((_ELT, 1, 128), jnp.float32),
    )(sel, pf, of, hb, lb)
    return out[:, 0, 0]


# speedup vs baseline: 70.2922x; 68.2653x over previous
import functools
import numpy as np
import jax
import jax.numpy as jnp
from jax.experimental import pallas as pl
from jax.experimental.pallas import tpu as pltpu

_ELT = 64
_ROW = 65536
_C = 1024
_R = _ELT * _ROW // _C
_RSEG = _ROW // _C
_T2 = np.float32((2.0 * np.sin(np.radians(10.0) / 2.0)) ** 2)


def _tc_body(p_ref, o_ref, h_ref, lam_ref, out_ref):
    d = p_ref[...] - o_ref[...]
    d2 = d * d
    s2 = d2[0] + d2[1] + d2[2]
    h = h_ref[0, 0, 0]
    lam = lam_ref[0, 0, 0]
    p = h * jnp.exp(s2 * (-1.0 / _T2) * lam) + (1.0 - h)
    log_p = jnp.where(s2 < _T2, jnp.log(p), 0.0)
    out_ref[...] = jnp.sum(log_p, dtype=jnp.float32)[None, None, None] * jnp.ones(
        (1, 1, 128), jnp.float32)


@jax.jit
def kernel(u_pred, u_obs, h, lam):
    pt = u_pred.T.reshape(3, _R, _C)
    ot = u_obs.T.reshape(3, _R, _C)
    hb = jnp.broadcast_to(h[:, None, None], (_ELT, 1, 128))
    lb = jnp.broadcast_to(lam[:, None, None], (_ELT, 1, 128))
    out = pl.pallas_call(
        _tc_body,
        grid=(_ELT,),
        in_specs=[
            pl.BlockSpec((3, _RSEG, _C), lambda e: (0, e, 0)),
            pl.BlockSpec((3, _RSEG, _C), lambda e: (0, e, 0)),
            pl.BlockSpec((1, 1, 128), lambda e: (e, 0, 0)),
            pl.BlockSpec((1, 1, 128), lambda e: (e, 0, 0)),
        ],
        out_specs=pl.BlockSpec((1, 1, 128), lambda e: (e, 0, 0)),
        out_shape=jax.ShapeDtypeStruct((_ELT, 1, 128), jnp.float32),
    )(pt, ot, hb, lb)
    return out[:, 0, 0]
